# baseline (speedup 1.0000x reference)
import functools

import jax
import jax.numpy as jnp
from jax import lax
from jax.experimental import pallas as pl
from jax.experimental.pallas import tpu as pltpu
from jax.experimental.pallas import tpu_sc as plsc

L = 16
NC = 2
NS = 16
NW = NC * NS

GATHER_ROWS = 128
GROUP = 10


@functools.lru_cache(maxsize=None)
def _make_sc_lookup(B, V, D):
    b_per_w = B // NW
    n_gathers = b_per_w // GATHER_ROWS
    n_groups = n_gathers // GROUP
    assert b_per_w * NW == B
    assert n_groups * GROUP == n_gathers
    group_rows = GROUP * GATHER_ROWS

    mesh = plsc.VectorSubcoreMesh(core_axis_name="c", subcore_axis_name="s")

    @functools.partial(
        pl.kernel,
        mesh=mesh,
        out_type=(
            jax.ShapeDtypeStruct((B, D), jnp.float32),
            jax.ShapeDtypeStruct((B,), jnp.float32),
        ),
        scratch_types=[
            pltpu.VMEM((b_per_w,), jnp.int32),
            pltpu.VMEM((b_per_w,), jnp.float32),
            pltpu.VMEM((group_rows, D), jnp.float32),
            pltpu.SemaphoreType.DMA,
        ],
        compiler_params=pltpu.CompilerParams(use_tc_tiling_on_sc=False),
    )
    def k(table_hbm, idx_hbm, res_hbm, mask_hbm, idx_v, mask_v, rows_v, sem):
        wid = lax.axis_index("s") * NC + lax.axis_index("c")
        base = wid * b_per_w
        pltpu.sync_copy(idx_hbm.at[pl.ds(base, b_per_w)], idx_v)

        def mask_body(i, carry):
            v = idx_v[pl.ds(i * L, L)]
            mask_v[pl.ds(i * L, L)] = jnp.where(
                v == 0, jnp.float32(1.0), jnp.float32(0.0))
            return carry
        lax.fori_loop(0, b_per_w // L, mask_body, 0)
        pltpu.sync_copy(mask_v, mask_hbm.at[pl.ds(base, b_per_w)])

        def group_body(g, carry):
            goff = g * group_rows
            copies = []
            for b in range(GROUP):
                src = table_hbm.at[
                    idx_v.at[pl.ds(goff + b * GATHER_ROWS, GATHER_ROWS)]]
                dst = rows_v.at[pl.ds(b * GATHER_ROWS, GATHER_ROWS)]
                copies.append(pltpu.async_copy(src, dst, sem))
            for c in copies:
                c.wait()
            pltpu.sync_copy(rows_v, res_hbm.at[pl.ds(base + goff, group_rows)])
            return carry
        lax.fori_loop(0, n_groups, group_body, 0)

    return k


def kernel(input, table):
    B = input.shape[0] * input.shape[1]
    V, D = table.shape
    idx = input.reshape(B)
    res, mask = _make_sc_lookup(B, V, D)(table, idx)
    return res.reshape(input.shape + (D,)), mask.reshape(input.shape)
